# baseline (speedup 1.0000x reference)
import jax
import jax.numpy as jnp
from jax import lax
from jax.experimental import pallas as pl
from jax.experimental.pallas import tpu as pltpu
from jax.experimental.pallas import tpu_sc as plsc

N = 100000
E = 1600000
C = 1000
D = 32
DH = 16
NS = 16
CH = 128
NCHT = 784
NCHUNKP = NCHT * NS
EP = NCHUNKP * CH
RW = 8
NB = 160
NBLK = N // NB


def _iota16():
    return lax.iota(jnp.int32, 16)


def _degrees_body(edges_hbm, deg2_hbm, deg_sp, idx0_v, idx1_v, idx2_v,
                  ones_v, dbuf_v, lsem0, lsem1, lsem2, ssem0, ssem1, ssem2):
    c = lax.axis_index("c")
    s = lax.axis_index("s")
    for g in range(CH // 16):
        ones_v[pl.ds(g * 16, 16)] = jnp.full((16,), 1.0, jnp.float32)
    for g in range(NB // 16):
        dbuf_v[pl.ds(g * 16, 16)] = jnp.zeros((16,), jnp.float32)

    @pl.loop(s, NBLK, step=NS)
    def _zero(blk):
        pltpu.sync_copy(dbuf_v, deg_sp.at[pl.ds(blk * NB, NB)])

    @pl.when(s == 0)
    def _zpad():
        pltpu.sync_copy(dbuf_v.at[pl.ds(0, 8)], deg_sp.at[pl.ds(N, 8)])

    plsc.subcore_barrier()

    ebase = c * EP
    bufs = ((idx0_v, lsem0, ssem0), (idx1_v, lsem1, ssem1),
            (idx2_v, lsem2, ssem2))

    def chunk_slice(k):
        return edges_hbm.at[pl.ds(ebase + (s + k * NS) * CH, CH)]

    def fire_lin(k, bset):
        pltpu.async_copy(chunk_slice(k), bset[0], bset[1])

    def wait_lin(k, bset):
        pltpu.make_async_copy(chunk_slice(k), bset[0], bset[1]).wait()

    def fire_scat(bset):
        pltpu.async_copy(ones_v, deg_sp.at[bset[0]], bset[2], add=True)

    def wait_scat(bset):
        pltpu.make_async_copy(ones_v, deg_sp.at[bset[0]], bset[2]).wait()

    def half_iter(k, p, fire_l2, wait_next=True):
        cur, nxt, nx2 = bufs[p % 3], bufs[(p + 1) % 3], bufs[(p + 2) % 3]
        if wait_next:
            wait_lin(k + 1, nxt)
        wait_scat(nx2)
        if fire_l2:
            fire_lin(k + 2, nx2)
        fire_scat(cur)

    pltpu.sync_copy(chunk_slice(0), idx0_v)
    fire_lin(1, bufs[1])
    wait_lin(1, bufs[1])
    fire_lin(2, bufs[2])
    fire_scat(bufs[0])
    wait_lin(2, bufs[2])
    wait_scat(bufs[0])
    fire_lin(3, bufs[0])
    fire_scat(bufs[1])

    @pl.loop(0, (NCHT - 4) // 3)
    def _trips(t):
        k = 2 + t * 3
        half_iter(k, 2, True)
        half_iter(k + 1, 0, True)
        half_iter(k + 2, 1, True)

    half_iter(NCHT - 2, 2, False)
    half_iter(NCHT - 1, 0, False, wait_next=False)
    wait_scat(bufs[0])

    plsc.subcore_barrier()

    nbase = c * N

    @pl.loop(s, NBLK, step=NS)
    def _writeout(blk):
        base = blk * NB
        pltpu.sync_copy(deg_sp.at[pl.ds(base, NB)], dbuf_v)
        pltpu.sync_copy(dbuf_v, deg2_hbm.at[pl.ds(nbase + base, NB)])


def _matmul_body(emb_ref, w_ref, deg_ref, out_ref, rdeg_ref):
    out_ref[...] = jnp.dot(emb_ref[...], w_ref[...],
                           preferred_element_type=jnp.float32)
    rdeg_ref[...] = lax.rsqrt(jnp.maximum(deg_ref[...], jnp.float32(1.0)))


def _conv_body(edges_hbm, w_hbm, rec_hbm, deg2_hbm, embw_hbm, b_hbm,
               out_hbm,
               acc_sp, embw_v, rdg_v, fbuf_v,
               src0_v, src1_v, src2_v, dst0_v, dst1_v, dst2_v,
               w0_v, w1_v, w2_v, rec0_v, rec1_v, rec2_v,
               rows0_v, rows1_v, rows2_v, bh_v,
               lsem0, lsem1, lsem2, rsem0, rsem1, rsem2,
               ssem0, ssem1, ssem2):
    c = lax.axis_index("c")
    s = lax.axis_index("s")
    iota = _iota16()

    pltpu.sync_copy(embw_hbm.at[c], embw_v)
    pltpu.sync_copy(b_hbm.at[c, 0], bh_v)

    for i in range(NB):
        fbuf_v[i] = jnp.zeros((16,), jnp.float32)

    @pl.loop(s, NBLK, step=NS)
    def _zero(blk):
        pltpu.sync_copy(fbuf_v, acc_sp.at[pl.ds(blk * NB, NB), :])

    @pl.when(s == 0)
    def _zpad():
        pltpu.sync_copy(fbuf_v.at[pl.ds(0, 8), :], acc_sp.at[pl.ds(N, 8), :])

    plsc.subcore_barrier()

    bufs = ((src0_v, dst0_v, w0_v, rec0_v, rows0_v, lsem0, rsem0, ssem0),
            (src1_v, dst1_v, w1_v, rec1_v, rows1_v, lsem1, rsem1, ssem1),
            (src2_v, dst2_v, w2_v, rec2_v, rows2_v, lsem2, rsem2, ssem2))

    def lin_descs(k, bset):
        eb = (s + k * NS) * CH
        return (
            (edges_hbm.at[pl.ds(eb, CH)], bset[0], bset[5]),
            (edges_hbm.at[pl.ds(EP + eb, CH)], bset[1], bset[5]),
            (w_hbm.at[pl.ds(eb, CH)], bset[2], bset[5]),
        )

    def fire_lin(k, bset):
        for sref, dref, sem in lin_descs(k, bset):
            pltpu.async_copy(sref, dref, sem)

    def wait_lin(k, bset):
        for sref, dref, sem in lin_descs(k, bset):
            pltpu.make_async_copy(sref, dref, sem).wait()

    def fire_rec(bset):
        pltpu.async_copy(rec_hbm.at[bset[0]], bset[3], bset[6])

    def wait_rec(bset):
        pltpu.make_async_copy(rec_hbm.at[bset[0]], bset[3], bset[6]).wait()

    def fire_scat(bset):
        pltpu.async_copy(bset[4], acc_sp.at[bset[1]], bset[7], add=True)

    def wait_scat(bset):
        pltpu.make_async_copy(bset[4], acc_sp.at[bset[1]], bset[7]).wait()

    zeros16 = jnp.zeros((16,), jnp.int32)
    ones16 = jnp.full((16,), 1, jnp.int32)

    def compute_rows(bset):
        w_v, rec_v, rows_v = bset[2], bset[3], bset[4]

        @pl.loop(0, CH // 16, unroll=2)
        def _grp(g):
            idx16 = g * 16 + iota
            rd16 = plsc.load_gather(rec_v, [idx16, zeros16])
            lab16 = lax.convert_element_type(
                plsc.load_gather(rec_v, [idx16, ones16]), jnp.int32)
            sv = w_v[pl.ds(g * 16, 16)] * rd16
            for j in range(DH):
                j16 = jnp.full((16,), j, jnp.int32)
                vals = plsc.load_gather(embw_v, [lab16, j16])
                plsc.store_scatter(rows_v, [idx16, j16], vals * sv)

    def half_iter(k, p, fire_r, fire_l2):
        cur, nxt, nx2 = bufs[p % 3], bufs[(p + 1) % 3], bufs[(p + 2) % 3]
        if fire_r:
            wait_lin(k + 1, nxt)
            fire_rec(nxt)
        wait_rec(cur)
        wait_scat(nx2)
        if fire_l2:
            fire_lin(k + 2, nx2)
        compute_rows(cur)
        fire_scat(cur)

    fire_lin(0, bufs[0])
    wait_lin(0, bufs[0])
    fire_rec(bufs[0])
    fire_lin(1, bufs[1])

    wait_lin(1, bufs[1])
    fire_rec(bufs[1])
    wait_rec(bufs[0])
    fire_lin(2, bufs[2])
    compute_rows(bufs[0])
    fire_scat(bufs[0])

    wait_lin(2, bufs[2])
    fire_rec(bufs[2])
    wait_rec(bufs[1])
    wait_scat(bufs[0])
    fire_lin(3, bufs[0])
    compute_rows(bufs[1])
    fire_scat(bufs[1])

    @pl.loop(0, (NCHT - 4) // 3)
    def _trips(t):
        k = 2 + t * 3
        half_iter(k, 2, True, True)
        half_iter(k + 1, 0, True, True)
        half_iter(k + 2, 1, True, True)

    half_iter(NCHT - 2, 2, True, False)
    half_iter(NCHT - 1, 0, False, False)
    wait_scat(bufs[0])

    plsc.subcore_barrier()

    bvec = bh_v[...]

    @pl.loop(s, NBLK, step=NS)
    def _final(blk):
        base = blk * NB
        pltpu.sync_copy(acc_sp.at[pl.ds(base, NB), :], fbuf_v)
        pltpu.sync_copy(deg2_hbm.at[pl.ds(N + base, NB)], rdg_v)
        for g in range(NB // 16):
            rv = rdg_v[pl.ds(g * 16, 16)]
            for i in range(16):
                n = g * 16 + i
                fbuf_v[n] = fbuf_v[n] * jnp.full((16,), rv[i], jnp.float32) + bvec
        pltpu.sync_copy(fbuf_v, out_hbm.at[pl.ds(base, NB), pl.ds(c * DH, DH)])


def kernel(node_labels, edge_index, edge_weight, emb_table, W, b):
    labels = node_labels.astype(jnp.int32)
    ei = edge_index.astype(jnp.int32)
    pad_e = EP - E
    edges_pad = jnp.concatenate(
        [ei, jnp.full((2, pad_e), N, jnp.int32)], axis=1).reshape(2 * EP)
    w_pad = jnp.concatenate(
        [edge_weight, jnp.zeros((pad_e,), jnp.float32)])
    mesh = plsc.VectorSubcoreMesh(core_axis_name="c", subcore_axis_name="s")
    scp = pltpu.CompilerParams(use_tc_tiling_on_sc=False,
                               needs_layout_passes=False)

    degraw = pl.kernel(
        _degrees_body,
        out_type=jax.ShapeDtypeStruct((2 * N,), jnp.float32),
        mesh=mesh,
        compiler_params=scp,
        scratch_types=[
            pltpu.VMEM_SHARED((N + 8,), jnp.float32),
            pltpu.VMEM((CH,), jnp.int32),
            pltpu.VMEM((CH,), jnp.int32),
            pltpu.VMEM((CH,), jnp.int32),
            pltpu.VMEM((CH,), jnp.float32),
            pltpu.VMEM((NB,), jnp.float32),
            pltpu.SemaphoreType.DMA,
            pltpu.SemaphoreType.DMA,
            pltpu.SemaphoreType.DMA,
            pltpu.SemaphoreType.DMA,
            pltpu.SemaphoreType.DMA,
            pltpu.SemaphoreType.DMA,
        ],
    )(edges_pad)

    embw, deg2 = pl.pallas_call(
        _matmul_body,
        out_shape=(
            jax.ShapeDtypeStruct((C, D), jnp.float32),
            jax.ShapeDtypeStruct((2 * N,), jnp.float32),
        ),
    )(emb_table, W, degraw)
    embw2 = embw.reshape(C, 2, DH).transpose(1, 0, 2)
    b3 = b.reshape(2, 1, DH)
    labf = labels.astype(jnp.float32)
    rec = jnp.pad(jnp.stack([deg2[:N], labf], axis=1),
                  ((0, 8), (0, RW - 2)))

    out = pl.kernel(
        _conv_body,
        out_type=jax.ShapeDtypeStruct((N, D), jnp.float32),
        mesh=mesh,
        compiler_params=scp,
        scratch_types=[
            pltpu.VMEM_SHARED((N + 8, DH), jnp.float32),
            pltpu.VMEM((C, DH), jnp.float32),
            pltpu.VMEM((NB,), jnp.float32),
            pltpu.VMEM((NB, DH), jnp.float32),
            pltpu.VMEM((CH,), jnp.int32),
            pltpu.VMEM((CH,), jnp.int32),
            pltpu.VMEM((CH,), jnp.int32),
            pltpu.VMEM((CH,), jnp.int32),
            pltpu.VMEM((CH,), jnp.int32),
            pltpu.VMEM((CH,), jnp.int32),
            pltpu.VMEM((CH,), jnp.float32),
            pltpu.VMEM((CH,), jnp.float32),
            pltpu.VMEM((CH,), jnp.float32),
            pltpu.VMEM((CH, RW), jnp.float32),
            pltpu.VMEM((CH, RW), jnp.float32),
            pltpu.VMEM((CH, RW), jnp.float32),
            pltpu.VMEM((CH, DH), jnp.float32),
            pltpu.VMEM((CH, DH), jnp.float32),
            pltpu.VMEM((CH, DH), jnp.float32),
            pltpu.VMEM((DH,), jnp.float32),
            pltpu.SemaphoreType.DMA,
            pltpu.SemaphoreType.DMA,
            pltpu.SemaphoreType.DMA,
            pltpu.SemaphoreType.DMA,
            pltpu.SemaphoreType.DMA,
            pltpu.SemaphoreType.DMA,
            pltpu.SemaphoreType.DMA,
            pltpu.SemaphoreType.DMA,
            pltpu.SemaphoreType.DMA,
        ],
    )(edges_pad, w_pad, rec, deg2, embw2, b3)
    return out


# speedup vs baseline: 1.0319x; 1.0319x over previous
import jax
import jax.numpy as jnp
from jax import lax
from jax.experimental import pallas as pl
from jax.experimental.pallas import tpu as pltpu
from jax.experimental.pallas import tpu_sc as plsc

N = 100000
E = 1600000
C = 1000
D = 32
DH = 16
NS = 16
CH = 128
NCHT = 784
NCHUNKP = NCHT * NS
EP = NCHUNKP * CH
RW = 8
NB = 160
NBLK = N // NB


def _iota16():
    return lax.iota(jnp.int32, 16)


def _degrees_body(edges_hbm, labels_hbm, deg2_hbm, rec_hbm, deg_sp,
                  idx0_v, idx1_v, idx2_v, ones_v, dbuf_v, lab_v, rec_v,
                  lsem0, lsem1, lsem2, ssem0, ssem1, ssem2):
    c = lax.axis_index("c")
    s = lax.axis_index("s")
    for g in range(CH // 16):
        ones_v[pl.ds(g * 16, 16)] = jnp.full((16,), 1.0, jnp.float32)
    for g in range(NB // 16):
        dbuf_v[pl.ds(g * 16, 16)] = jnp.zeros((16,), jnp.float32)

    @pl.loop(s, NBLK, step=NS)
    def _zero(blk):
        pltpu.sync_copy(dbuf_v, deg_sp.at[pl.ds(blk * NB, NB)])

    @pl.when(s == 0)
    def _zpad():
        pltpu.sync_copy(dbuf_v.at[pl.ds(0, 8)], deg_sp.at[pl.ds(N, 8)])

    plsc.subcore_barrier()

    ebase = c * EP
    bufs = ((idx0_v, lsem0, ssem0), (idx1_v, lsem1, ssem1),
            (idx2_v, lsem2, ssem2))

    def chunk_slice(k):
        return edges_hbm.at[pl.ds(ebase + (s + k * NS) * CH, CH)]

    def fire_lin(k, bset):
        pltpu.async_copy(chunk_slice(k), bset[0], bset[1])

    def wait_lin(k, bset):
        pltpu.make_async_copy(chunk_slice(k), bset[0], bset[1]).wait()

    def fire_scat(bset):
        pltpu.async_copy(ones_v, deg_sp.at[bset[0]], bset[2], add=True)

    def wait_scat(bset):
        pltpu.make_async_copy(ones_v, deg_sp.at[bset[0]], bset[2]).wait()

    def half_iter(k, p, fire_l2, wait_next=True):
        cur, nxt, nx2 = bufs[p % 3], bufs[(p + 1) % 3], bufs[(p + 2) % 3]
        if wait_next:
            wait_lin(k + 1, nxt)
        wait_scat(nx2)
        if fire_l2:
            fire_lin(k + 2, nx2)
        fire_scat(cur)

    pltpu.sync_copy(chunk_slice(0), idx0_v)
    fire_lin(1, bufs[1])
    wait_lin(1, bufs[1])
    fire_lin(2, bufs[2])
    fire_scat(bufs[0])
    wait_lin(2, bufs[2])
    wait_scat(bufs[0])
    fire_lin(3, bufs[0])
    fire_scat(bufs[1])

    @pl.loop(0, (NCHT - 4) // 3)
    def _trips(t):
        k = 2 + t * 3
        half_iter(k, 2, True)
        half_iter(k + 1, 0, True)
        half_iter(k + 2, 1, True)

    half_iter(NCHT - 2, 2, False)
    half_iter(NCHT - 1, 0, False, wait_next=False)
    wait_scat(bufs[0])

    plsc.subcore_barrier()

    nbase = c * N
    iota = _iota16()
    zeros16 = jnp.zeros((16,), jnp.int32)
    ones16i = jnp.full((16,), 1, jnp.int32)

    def _rsqrt16(x):
        i = plsc.bitcast(x, jnp.int32)
        i = jnp.int32(0x5F3759DF) - lax.shift_right_arithmetic(i, 1)
        y = plsc.bitcast(i, jnp.float32)
        xh = x * jnp.float32(0.5)
        for _ in range(3):
            y = y * (jnp.float32(1.5) - xh * y * y)
        return y

    @pl.loop(s, NBLK, step=NS)
    def _writeout(blk):
        base = blk * NB
        pltpu.sync_copy(deg_sp.at[pl.ds(base, NB)], dbuf_v)
        pltpu.sync_copy(labels_hbm.at[pl.ds(base, NB)], lab_v)
        for g in range(NB // 16):
            v = jnp.maximum(dbuf_v[pl.ds(g * 16, 16)], jnp.float32(1.0))
            dbuf_v[pl.ds(g * 16, 16)] = _rsqrt16(v)
        pltpu.sync_copy(dbuf_v, deg2_hbm.at[pl.ds(nbase + base, NB)])

        @pl.when(c == 0)
        def _rec():
            for g in range(NB // 16):
                n16 = g * 16 + iota
                rd16 = dbuf_v[pl.ds(g * 16, 16)]
                lf16 = lax.convert_element_type(lab_v[pl.ds(g * 16, 16)],
                                                jnp.float32)
                plsc.store_scatter(rec_v, [n16, zeros16], rd16)
                plsc.store_scatter(rec_v, [n16, ones16i], lf16)
            pltpu.sync_copy(rec_v, rec_hbm.at[pl.ds(base, NB), :])


def _matmul_body(emb_ref, w_ref, out_ref):
    out_ref[...] = jnp.dot(emb_ref[...], w_ref[...],
                           preferred_element_type=jnp.float32)


def _conv_body(edges_hbm, w_hbm, rec_hbm, deg2_hbm, embw_hbm, b_hbm,
               out_hbm,
               acc_sp, embw_v, rdg_v, fbuf_v,
               src0_v, src1_v, src2_v, dst0_v, dst1_v, dst2_v,
               w0_v, w1_v, w2_v, rec0_v, rec1_v, rec2_v,
               rows0_v, rows1_v, rows2_v, bh_v,
               lsem0, lsem1, lsem2, rsem0, rsem1, rsem2,
               ssem0, ssem1, ssem2):
    c = lax.axis_index("c")
    s = lax.axis_index("s")
    iota = _iota16()

    pltpu.sync_copy(embw_hbm.at[c], embw_v)
    pltpu.sync_copy(b_hbm.at[c, 0], bh_v)

    for i in range(NB):
        fbuf_v[i] = jnp.zeros((16,), jnp.float32)

    @pl.loop(s, NBLK, step=NS)
    def _zero(blk):
        pltpu.sync_copy(fbuf_v, acc_sp.at[pl.ds(blk * NB, NB), :])

    @pl.when(s == 0)
    def _zpad():
        pltpu.sync_copy(fbuf_v.at[pl.ds(0, 8), :], acc_sp.at[pl.ds(N, 8), :])

    plsc.subcore_barrier()

    bufs = ((src0_v, dst0_v, w0_v, rec0_v, rows0_v, lsem0, rsem0, ssem0),
            (src1_v, dst1_v, w1_v, rec1_v, rows1_v, lsem1, rsem1, ssem1),
            (src2_v, dst2_v, w2_v, rec2_v, rows2_v, lsem2, rsem2, ssem2))

    def lin_descs(k, bset):
        eb = (s + k * NS) * CH
        return (
            (edges_hbm.at[pl.ds(eb, CH)], bset[0], bset[5]),
            (edges_hbm.at[pl.ds(EP + eb, CH)], bset[1], bset[5]),
            (w_hbm.at[pl.ds(eb, CH)], bset[2], bset[5]),
        )

    def fire_lin(k, bset):
        for sref, dref, sem in lin_descs(k, bset):
            pltpu.async_copy(sref, dref, sem)

    def wait_lin(k, bset):
        for sref, dref, sem in lin_descs(k, bset):
            pltpu.make_async_copy(sref, dref, sem).wait()

    def fire_rec(bset):
        pltpu.async_copy(rec_hbm.at[bset[0]], bset[3], bset[6])

    def wait_rec(bset):
        pltpu.make_async_copy(rec_hbm.at[bset[0]], bset[3], bset[6]).wait()

    def fire_scat(bset):
        pltpu.async_copy(bset[4], acc_sp.at[bset[1]], bset[7], add=True)

    def wait_scat(bset):
        pltpu.make_async_copy(bset[4], acc_sp.at[bset[1]], bset[7]).wait()

    zeros16 = jnp.zeros((16,), jnp.int32)
    ones16 = jnp.full((16,), 1, jnp.int32)

    def compute_rows(bset):
        w_v, rec_v, rows_v = bset[2], bset[3], bset[4]

        @pl.loop(0, CH // 16, unroll=2)
        def _grp(g):
            idx16 = g * 16 + iota
            rd16 = plsc.load_gather(rec_v, [idx16, zeros16])
            lab16 = lax.convert_element_type(
                plsc.load_gather(rec_v, [idx16, ones16]), jnp.int32)
            lab16 = jnp.clip(lab16, 0, C - 1)
            sv = w_v[pl.ds(g * 16, 16)] * rd16
            for j in range(DH):
                j16 = jnp.full((16,), j, jnp.int32)
                vals = plsc.load_gather(embw_v, [lab16, j16])
                plsc.store_scatter(rows_v, [idx16, j16], vals * sv)

    def half_iter(k, p, fire_r, fire_l2):
        cur, nxt, nx2 = bufs[p % 3], bufs[(p + 1) % 3], bufs[(p + 2) % 3]
        if fire_r:
            wait_lin(k + 1, nxt)
            fire_rec(nxt)
        wait_rec(cur)
        wait_scat(nx2)
        if fire_l2:
            fire_lin(k + 2, nx2)
        compute_rows(cur)
        fire_scat(cur)

    fire_lin(0, bufs[0])
    wait_lin(0, bufs[0])
    fire_rec(bufs[0])
    fire_lin(1, bufs[1])

    wait_lin(1, bufs[1])
    fire_rec(bufs[1])
    wait_rec(bufs[0])
    fire_lin(2, bufs[2])
    compute_rows(bufs[0])
    fire_scat(bufs[0])

    wait_lin(2, bufs[2])
    fire_rec(bufs[2])
    wait_rec(bufs[1])
    wait_scat(bufs[0])
    fire_lin(3, bufs[0])
    compute_rows(bufs[1])
    fire_scat(bufs[1])

    @pl.loop(0, (NCHT - 4) // 3)
    def _trips(t):
        k = 2 + t * 3
        half_iter(k, 2, True, True)
        half_iter(k + 1, 0, True, True)
        half_iter(k + 2, 1, True, True)

    half_iter(NCHT - 2, 2, True, False)
    half_iter(NCHT - 1, 0, False, False)
    wait_scat(bufs[0])

    plsc.subcore_barrier()

    bvec = bh_v[...]

    @pl.loop(s, NBLK, step=NS)
    def _final(blk):
        base = blk * NB
        pltpu.sync_copy(acc_sp.at[pl.ds(base, NB), :], fbuf_v)
        pltpu.sync_copy(deg2_hbm.at[pl.ds(N + base, NB)], rdg_v)
        for g in range(NB // 16):
            rv = rdg_v[pl.ds(g * 16, 16)]
            for i in range(16):
                n = g * 16 + i
                fbuf_v[n] = fbuf_v[n] * jnp.full((16,), rv[i], jnp.float32) + bvec
        pltpu.sync_copy(fbuf_v, out_hbm.at[pl.ds(base, NB), pl.ds(c * DH, DH)])


def kernel(node_labels, edge_index, edge_weight, emb_table, W, b):
    labels_pad = jnp.concatenate(
        [node_labels.astype(jnp.int32), jnp.zeros((8,), jnp.int32)])
    ei = edge_index.astype(jnp.int32)
    pad_e = EP - E
    edges_pad = jnp.concatenate(
        [ei, jnp.full((2, pad_e), N, jnp.int32)], axis=1).reshape(2 * EP)
    w_pad = jnp.concatenate(
        [edge_weight, jnp.zeros((pad_e,), jnp.float32)])
    mesh = plsc.VectorSubcoreMesh(core_axis_name="c", subcore_axis_name="s")
    scp = pltpu.CompilerParams(use_tc_tiling_on_sc=False,
                               needs_layout_passes=False)

    deg2, rec = pl.kernel(
        _degrees_body,
        out_type=(
            jax.ShapeDtypeStruct((2 * N,), jnp.float32),
            jax.ShapeDtypeStruct((N + 8, RW), jnp.float32),
        ),
        mesh=mesh,
        compiler_params=scp,
        scratch_types=[
            pltpu.VMEM_SHARED((N + 8,), jnp.float32),
            pltpu.VMEM((CH,), jnp.int32),
            pltpu.VMEM((CH,), jnp.int32),
            pltpu.VMEM((CH,), jnp.int32),
            pltpu.VMEM((CH,), jnp.float32),
            pltpu.VMEM((NB,), jnp.float32),
            pltpu.VMEM((NB,), jnp.int32),
            pltpu.VMEM((NB, RW), jnp.float32),
            pltpu.SemaphoreType.DMA,
            pltpu.SemaphoreType.DMA,
            pltpu.SemaphoreType.DMA,
            pltpu.SemaphoreType.DMA,
            pltpu.SemaphoreType.DMA,
            pltpu.SemaphoreType.DMA,
        ],
    )(edges_pad, labels_pad)

    embw = pl.pallas_call(
        _matmul_body,
        out_shape=jax.ShapeDtypeStruct((C, D), jnp.float32),
    )(emb_table, W)
    embw2 = embw.reshape(C, 2, DH).transpose(1, 0, 2)
    b3 = b.reshape(2, 1, DH)

    out = pl.kernel(
        _conv_body,
        out_type=jax.ShapeDtypeStruct((N, D), jnp.float32),
        mesh=mesh,
        compiler_params=scp,
        scratch_types=[
            pltpu.VMEM_SHARED((N + 8, DH), jnp.float32),
            pltpu.VMEM((C, DH), jnp.float32),
            pltpu.VMEM((NB,), jnp.float32),
            pltpu.VMEM((NB, DH), jnp.float32),
            pltpu.VMEM((CH,), jnp.int32),
            pltpu.VMEM((CH,), jnp.int32),
            pltpu.VMEM((CH,), jnp.int32),
            pltpu.VMEM((CH,), jnp.int32),
            pltpu.VMEM((CH,), jnp.int32),
            pltpu.VMEM((CH,), jnp.int32),
            pltpu.VMEM((CH,), jnp.float32),
            pltpu.VMEM((CH,), jnp.float32),
            pltpu.VMEM((CH,), jnp.float32),
            pltpu.VMEM((CH, RW), jnp.float32),
            pltpu.VMEM((CH, RW), jnp.float32),
            pltpu.VMEM((CH, RW), jnp.float32),
            pltpu.VMEM((CH, DH), jnp.float32),
            pltpu.VMEM((CH, DH), jnp.float32),
            pltpu.VMEM((CH, DH), jnp.float32),
            pltpu.VMEM((DH,), jnp.float32),
            pltpu.SemaphoreType.DMA,
            pltpu.SemaphoreType.DMA,
            pltpu.SemaphoreType.DMA,
            pltpu.SemaphoreType.DMA,
            pltpu.SemaphoreType.DMA,
            pltpu.SemaphoreType.DMA,
            pltpu.SemaphoreType.DMA,
            pltpu.SemaphoreType.DMA,
            pltpu.SemaphoreType.DMA,
        ],
    )(edges_pad, w_pad, rec, deg2, embw2, b3)
    return out
